# baseline (speedup 1.0000x reference)
import jax
import jax.numpy as jnp
from jax.experimental import pallas as pl
from jax.experimental.pallas import tpu as pltpu

B = 128
D = 768
V = 27623
K = 256
VB = 1024
NB = 27
VP = NB * VB

_DN = (((1,), (1,)), ((), ()))


def _softplus(x):
    return jnp.maximum(x, 0.0) + jnp.log1p(jnp.exp(-jnp.abs(x)))


def _mm_kernel(h_ref, wp_ref, bp_ref, wm_ref, bm_ref, up_ref, um_ref):
    j = pl.program_id(0)
    h = h_ref[...]
    sp = jax.lax.dot_general(h, wp_ref[...], _DN,
                             preferred_element_type=jnp.float32) + bp_ref[...]
    sm = jax.lax.dot_general(h, wm_ref[...], _DN,
                             preferred_element_type=jnp.float32) + bm_ref[...]
    up = _softplus(sp)
    um = _softplus(sm)

    @pl.when(j == NB - 1)
    def _():
        lane = jax.lax.broadcasted_iota(jnp.int32, (B, VB), 1) + j * VB
        valid = lane < V
        up_ref[...] = jnp.where(valid, up, 0.0)
        um_ref[...] = jnp.where(valid, um, 0.0)

    @pl.when(j < NB - 1)
    def _():
        up_ref[...] = up
        um_ref[...] = um


def _kth_thresh(u):
    def body(_, carry):
        lo, hi = carry
        mid = lo + (hi - lo) // 2
        t = jax.lax.bitcast_convert_type(mid, jnp.float32)
        cnt = jnp.sum((u >= t).astype(jnp.int32), axis=1, keepdims=True)
        ge = cnt >= K
        return jnp.where(ge, mid, lo), jnp.where(ge, hi, mid)

    lo0 = jnp.zeros((B, 1), jnp.int32)
    hi0 = jnp.full((B, 1), 0x7F800000, jnp.int32)
    lo, _ = jax.lax.fori_loop(0, 31, body, (lo0, hi0))
    return jax.lax.bitcast_convert_type(lo, jnp.float32)


def _sel_kernel(up_ref, um_ref, sr_ref, wd_ref,
                z_ref, sq_ref, dsp_ref, dsm_ref,
                tp_ref, tm_ref, zacc_ref):
    j = pl.program_id(0)

    @pl.when(j == 0)
    def _():
        tp_ref[...] = _kth_thresh(up_ref[...])
        tm_ref[...] = _kth_thresh(um_ref[...])
        zacc_ref[...] = jnp.zeros_like(zacc_ref)

    u_p = up_ref[:, pl.ds(j * VB, VB)]
    u_m = um_ref[:, pl.ds(j * VB, VB)]
    dsp = jnp.where(u_p >= tp_ref[...], u_p, 0.0)
    dsm = jnp.where(u_m >= tm_ref[...], u_m, 0.0)

    last = j == NB - 1

    def mk_sq(sr):
        return jnp.maximum(sr + dsp, 0.0) - dsm

    @pl.when(jnp.logical_not(last))
    def _():
        sq = mk_sq(sr_ref[...])
        sq_ref[...] = sq
        zacc_ref[...] += jax.lax.dot_general(sq, wd_ref[...], _DN,
                                             preferred_element_type=jnp.float32)

    @pl.when(last)
    def _():
        lane = jax.lax.broadcasted_iota(jnp.int32, (B, VB), 1) + j * VB
        sr = jnp.where(lane < V, sr_ref[...], 0.0)
        sq = mk_sq(sr)
        sq_ref[...] = sq
        wlane = jax.lax.broadcasted_iota(jnp.int32, (D, VB), 1) + j * VB
        wd = jnp.where(wlane < V, wd_ref[...], 0.0)
        z = zacc_ref[...] + jax.lax.dot_general(sq, wd, _DN,
                                                preferred_element_type=jnp.float32)
        n = jnp.sqrt(jnp.sum(z * z, axis=1, keepdims=True))
        z_ref[...] = z / (n + 1e-6)

    dsp_ref[...] = dsp
    dsm_ref[...] = dsm


def kernel(h_t, sr_plus, sr_minus, W_plus, b_plus, W_minus, b_minus, W_dec):
    bp = b_plus[None, :]
    bm = b_minus[None, :]
    up, um = pl.pallas_call(
        _mm_kernel,
        grid=(NB,),
        in_specs=[
            pl.BlockSpec((B, D), lambda j: (0, 0)),
            pl.BlockSpec((VB, D), lambda j: (j, 0)),
            pl.BlockSpec((1, VB), lambda j: (0, j)),
            pl.BlockSpec((VB, D), lambda j: (j, 0)),
            pl.BlockSpec((1, VB), lambda j: (0, j)),
        ],
        out_specs=[
            pl.BlockSpec((B, VB), lambda j: (0, j)),
            pl.BlockSpec((B, VB), lambda j: (0, j)),
        ],
        out_shape=[jax.ShapeDtypeStruct((B, VP), jnp.float32)] * 2,
    )(h_t, W_plus, bp, W_minus, bm)

    z_hat, sq, ds_plus, ds_minus = pl.pallas_call(
        _sel_kernel,
        grid=(NB,),
        in_specs=[
            pl.BlockSpec((B, VP), lambda j: (0, 0)),
            pl.BlockSpec((B, VP), lambda j: (0, 0)),
            pl.BlockSpec((B, VB), lambda j: (0, j)),
            pl.BlockSpec((D, VB), lambda j: (0, j)),
        ],
        out_specs=[
            pl.BlockSpec((B, D), lambda j: (0, 0)),
            pl.BlockSpec((B, VB), lambda j: (0, j)),
            pl.BlockSpec((B, VB), lambda j: (0, j)),
            pl.BlockSpec((B, VB), lambda j: (0, j)),
        ],
        out_shape=[
            jax.ShapeDtypeStruct((B, D), jnp.float32),
            jax.ShapeDtypeStruct((B, V), jnp.float32),
            jax.ShapeDtypeStruct((B, V), jnp.float32),
            jax.ShapeDtypeStruct((B, V), jnp.float32),
        ],
        scratch_shapes=[
            pltpu.VMEM((B, 1), jnp.float32),
            pltpu.VMEM((B, 1), jnp.float32),
            pltpu.VMEM((B, D), jnp.float32),
        ],
    )(up, um, sr_plus, W_dec)

    return (z_hat, sq, ds_plus, ds_minus)


# speedup vs baseline: 1.0197x; 1.0197x over previous
import jax
import jax.numpy as jnp
from jax.experimental import pallas as pl
from jax.experimental.pallas import tpu as pltpu

B = 128
D = 768
V = 27623
K = 256
VB1 = 2304
NB1 = 12
VB = 1024
NB = 27
VP = NB * VB

_DN = (((1,), (1,)), ((), ()))


def _softplus(x):
    return jnp.maximum(x, 0.0) + jnp.log1p(jnp.exp(-jnp.abs(x)))


def _mm_kernel(h_ref, wp_ref, bp_ref, wm_ref, bm_ref, up_ref, um_ref):
    j = pl.program_id(0)
    h = h_ref[...]
    sp = jax.lax.dot_general(h, wp_ref[...], _DN,
                             preferred_element_type=jnp.float32) + bp_ref[...]
    sm = jax.lax.dot_general(h, wm_ref[...], _DN,
                             preferred_element_type=jnp.float32) + bm_ref[...]
    up = _softplus(sp)
    um = _softplus(sm)

    @pl.when(j == NB1 - 1)
    def _():
        lane = jax.lax.broadcasted_iota(jnp.int32, (B, VB1), 1) + j * VB1
        valid = lane < V
        up_ref[...] = jnp.where(valid, up, 0.0)
        um_ref[...] = jnp.where(valid, um, 0.0)

    @pl.when(j < NB1 - 1)
    def _():
        up_ref[...] = up
        um_ref[...] = um


def _kth_thresh(u):
    def body(_, carry):
        lo, hi = carry
        mid = lo + (hi - lo) // 2
        t = jax.lax.bitcast_convert_type(mid, jnp.float32)
        cnt = jnp.sum((u >= t).astype(jnp.int32), axis=1, keepdims=True)
        ge = cnt >= K
        return jnp.where(ge, mid, lo), jnp.where(ge, hi, mid)

    lo0 = jnp.zeros((B, 1), jnp.int32)
    hi0 = jnp.full((B, 1), 0x7F800000, jnp.int32)
    lo, _ = jax.lax.fori_loop(0, 31, body, (lo0, hi0))
    return jax.lax.bitcast_convert_type(lo, jnp.float32)


def _sel_kernel(up_ref, um_ref, sr_ref, wd_ref,
                z_ref, sq_ref, dsp_ref, dsm_ref,
                tp_ref, tm_ref, zacc_ref):
    j = pl.program_id(0)

    @pl.when(j == 0)
    def _():
        tp_ref[...] = _kth_thresh(up_ref[...])
        tm_ref[...] = _kth_thresh(um_ref[...])
        zacc_ref[...] = jnp.zeros_like(zacc_ref)

    u_p = up_ref[:, pl.ds(j * VB, VB)]
    u_m = um_ref[:, pl.ds(j * VB, VB)]
    dsp = jnp.where(u_p >= tp_ref[...], u_p, 0.0)
    dsm = jnp.where(u_m >= tm_ref[...], u_m, 0.0)

    last = j == NB - 1

    def mk_sq(sr):
        return jnp.maximum(sr + dsp, 0.0) - dsm

    @pl.when(jnp.logical_not(last))
    def _():
        sq = mk_sq(sr_ref[...])
        sq_ref[...] = sq
        zacc_ref[...] += jax.lax.dot_general(sq, wd_ref[...], _DN,
                                             preferred_element_type=jnp.float32)

    @pl.when(last)
    def _():
        lane = jax.lax.broadcasted_iota(jnp.int32, (B, VB), 1) + j * VB
        sr = jnp.where(lane < V, sr_ref[...], 0.0)
        sq = mk_sq(sr)
        sq_ref[...] = sq
        wlane = jax.lax.broadcasted_iota(jnp.int32, (D, VB), 1) + j * VB
        wd = jnp.where(wlane < V, wd_ref[...], 0.0)
        z = zacc_ref[...] + jax.lax.dot_general(sq, wd, _DN,
                                                preferred_element_type=jnp.float32)
        n = jnp.sqrt(jnp.sum(z * z, axis=1, keepdims=True))
        z_ref[...] = z / (n + 1e-6)

    dsp_ref[...] = dsp
    dsm_ref[...] = dsm


def kernel(h_t, sr_plus, sr_minus, W_plus, b_plus, W_minus, b_minus, W_dec):
    bp = b_plus[None, :]
    bm = b_minus[None, :]
    up, um = pl.pallas_call(
        _mm_kernel,
        grid=(NB1,),
        in_specs=[
            pl.BlockSpec((B, D), lambda j: (0, 0)),
            pl.BlockSpec((VB1, D), lambda j: (j, 0)),
            pl.BlockSpec((1, VB1), lambda j: (0, j)),
            pl.BlockSpec((VB1, D), lambda j: (j, 0)),
            pl.BlockSpec((1, VB1), lambda j: (0, j)),
        ],
        out_specs=[
            pl.BlockSpec((B, VB1), lambda j: (0, j)),
            pl.BlockSpec((B, VB1), lambda j: (0, j)),
        ],
        out_shape=[jax.ShapeDtypeStruct((B, VP), jnp.float32)] * 2,
    )(h_t, W_plus, bp, W_minus, bm)

    z_hat, sq, ds_plus, ds_minus = pl.pallas_call(
        _sel_kernel,
        grid=(NB,),
        in_specs=[
            pl.BlockSpec((B, VP), lambda j: (0, 0)),
            pl.BlockSpec((B, VP), lambda j: (0, 0)),
            pl.BlockSpec((B, VB), lambda j: (0, j)),
            pl.BlockSpec((D, VB), lambda j: (0, j)),
        ],
        out_specs=[
            pl.BlockSpec((B, D), lambda j: (0, 0)),
            pl.BlockSpec((B, VB), lambda j: (0, j)),
            pl.BlockSpec((B, VB), lambda j: (0, j)),
            pl.BlockSpec((B, VB), lambda j: (0, j)),
        ],
        out_shape=[
            jax.ShapeDtypeStruct((B, D), jnp.float32),
            jax.ShapeDtypeStruct((B, V), jnp.float32),
            jax.ShapeDtypeStruct((B, V), jnp.float32),
            jax.ShapeDtypeStruct((B, V), jnp.float32),
        ],
        scratch_shapes=[
            pltpu.VMEM((B, 1), jnp.float32),
            pltpu.VMEM((B, 1), jnp.float32),
            pltpu.VMEM((B, D), jnp.float32),
        ],
    )(up, um, sr_plus, W_dec)

    return (z_hat, sq, ds_plus, ds_minus)
